# baseline (speedup 1.0000x reference)
import functools

import jax
import jax.numpy as jnp
from jax import lax
from jax.experimental import pallas as pl
from jax.experimental.pallas import tpu as pltpu
from jax.experimental.pallas import tpu_sc as plsc

_NUM_CORES = 2
_NUM_SUBCORES = 16
_NUM_WORKERS = _NUM_CORES * _NUM_SUBCORES
_LANES = 16
_NBUF = 2
_H0 = 128


def _emb_body(n, d, seq_per_w, n_pad,
              xt_hbm, tbl_hbm, pos_hbm, out_hbm,
              xbuf_v, poff_v, gidx_v, rows_v, obuf_v, pos_v,
              gsem0, gsem1, ssem0, ssem1):
  c = lax.axis_index("c")
  s = lax.axis_index("s")
  wid = s * _NUM_CORES + c
  base_seq = wid * seq_per_w
  gsems = (gsem0, gsem1)
  ssems = (ssem0, ssem1)
  n_outer = seq_per_w // _NBUF
  h1 = n - _H0
  nh = n // 2
  nblk = (n + _LANES - 1) // _LANES
  iota = lax.iota(jnp.int32, _LANES)

  pltpu.sync_copy(pos_hbm, pos_v)
  pltpu.sync_copy(xt_hbm.at[:, pl.ds(wid * seq_per_w, seq_per_w)],
                  xbuf_v.at[pl.ds(0, n)])

  def issue_gather(i_local, b):
    col = jnp.full((_LANES,), i_local, jnp.int32)
    for k in range(nblk):
      vals = plsc.load_gather(xbuf_v, [jnp.int32(_LANES * k) + iota, col])
      poff_v.at[b][pl.ds(_LANES * k, _LANES)] = (vals & 1) * d
      gidx_v.at[b][pl.ds(_LANES * k, _LANES)] = lax.shift_right_logical(vals, 1)
    rows_b = rows_v.at[b]
    pltpu.async_copy(tbl_hbm.at[gidx_v.at[b, pl.ds(0, _H0)]],
                     rows_b.at[pl.ds(0, _H0)], gsems[b])
    pltpu.async_copy(tbl_hbm.at[gidx_v.at[b, pl.ds(_H0, h1)]],
                     rows_b.at[pl.ds(_H0, h1)], gsems[b])

  def drain_gather(b):
    pltpu.make_async_copy(tbl_hbm.at[pl.ds(0, n)], rows_v.at[b],
                          gsems[b]).wait()

  def drain_scatter(b):
    pltpu.make_async_copy(obuf_v.at[b], out_hbm.at[pl.ds(0, nh)],
                          ssems[b]).wait()

  for b in range(_NBUF):
    issue_gather(jnp.int32(b), b)

  @pl.loop(0, n_outer)
  def _outer(o):
    for b in range(_NBUF):
      i_local = o * _NBUF + b
      @pl.when(o >= 1)
      def _():
        drain_scatter(b)
      drain_gather(b)

      @plsc.parallel_loop(0, nh, unroll=2)
      def _pair(q):
        offs = poff_v.at[b][pl.ds(2 * q, _LANES)]
        for h in range(2):
          j = 2 * q + h
          off = offs[h]
          for k in range(d // _LANES):
            obuf_v.at[b][q, pl.ds(h * d + k * _LANES, _LANES)] = (
                rows_v.at[b][j, pl.ds(off + k * _LANES, _LANES)]
                + pos_v[j, pl.ds(k * _LANES, _LANES)])

      @pl.when(o < n_outer - 1)
      def _():
        issue_gather(i_local + _NBUF, b)
      pltpu.async_copy(obuf_v.at[b],
                       out_hbm.at[pl.ds((base_seq + i_local) * nh, nh)],
                       ssems[b])

  for b in range(_NBUF):
    drain_scatter(b)


def kernel(X, token_table, pos_table):
  b, n = X.shape
  v, d = token_table.shape
  assert b % (_NUM_WORKERS * _NBUF) == 0 and d % _LANES == 0
  assert v % 2 == 0 and n % 2 == 0
  seq_per_w = b // _NUM_WORKERS
  assert _H0 <= n < 2 * _H0
  n_pad = ((n + _LANES - 1) // _LANES) * _LANES + _LANES

  xt = X.T.astype(jnp.int32)
  tbl2 = token_table.reshape(v // 2, 2 * d)
  mesh = plsc.VectorSubcoreMesh(core_axis_name="c", subcore_axis_name="s")

  emb = pl.kernel(
      functools.partial(_emb_body, n, d, seq_per_w, n_pad),
      out_type=jax.ShapeDtypeStruct((b * n // 2, 2 * d), jnp.float32),
      mesh=mesh,
      scratch_types=[
          pltpu.VMEM((n_pad, seq_per_w), jnp.int32),
          pltpu.VMEM((_NBUF, n_pad), jnp.int32),
          pltpu.VMEM((_NBUF, n_pad), jnp.int32),
          pltpu.VMEM((_NBUF, n, 2 * d), jnp.float32),
          pltpu.VMEM((_NBUF, n // 2, 2 * d), jnp.float32),
          pltpu.VMEM((n, d), jnp.float32),
          pltpu.SemaphoreType.DMA,
          pltpu.SemaphoreType.DMA,
          pltpu.SemaphoreType.DMA,
          pltpu.SemaphoreType.DMA,
      ],
      compiler_params=pltpu.CompilerParams(use_tc_tiling_on_sc=False,
                                           needs_layout_passes=False),
  )
  out = emb(xt, tbl2, pos_table)
  return out.reshape(b, n, d)


# speedup vs baseline: 1.3954x; 1.3954x over previous
import functools

import jax
import jax.numpy as jnp
from jax import lax
from jax.experimental import pallas as pl
from jax.experimental.pallas import tpu as pltpu
from jax.experimental.pallas import tpu_sc as plsc

_NUM_CORES = 2
_NUM_SUBCORES = 16
_NUM_WORKERS = _NUM_CORES * _NUM_SUBCORES
_LANES = 16
_NBUF = 2
_H0 = 128


def _emb_body(n, d, seq_per_w, n_pad,
              xt_hbm, tbl_hbm, pos_hbm, out_hbm,
              xbuf_v, gidx_v, rows_v, obuf_v, pos_v,
              gsem0, gsem1, ssem0, ssem1):
  c = lax.axis_index("c")
  s = lax.axis_index("s")
  wid = s * _NUM_CORES + c
  base_seq = wid * seq_per_w
  gsems = (gsem0, gsem1)
  ssems = (ssem0, ssem1)
  n_outer = seq_per_w // _NBUF
  h1 = n - _H0
  nblk = (n + _LANES - 1) // _LANES
  iota = lax.iota(jnp.int32, _LANES)

  pltpu.sync_copy(pos_hbm, pos_v)
  pltpu.sync_copy(xt_hbm.at[:, pl.ds(wid * seq_per_w, seq_per_w)],
                  xbuf_v.at[pl.ds(0, n)])

  def issue_gather(i_local, b):
    col = jnp.full((_LANES,), i_local, jnp.int32)
    for k in range(nblk):
      vals = plsc.load_gather(xbuf_v, [jnp.int32(_LANES * k) + iota, col])
      gidx_v.at[b][pl.ds(_LANES * k, _LANES)] = vals
    rows_b = rows_v.at[b]
    pltpu.async_copy(tbl_hbm.at[gidx_v.at[b, pl.ds(0, _H0)]],
                     rows_b.at[pl.ds(0, _H0)], gsems[b])
    pltpu.async_copy(tbl_hbm.at[gidx_v.at[b, pl.ds(_H0, h1)]],
                     rows_b.at[pl.ds(_H0, h1)], gsems[b])

  def drain_gather(b):
    pltpu.make_async_copy(tbl_hbm.at[pl.ds(0, n)], rows_v.at[b],
                          gsems[b]).wait()

  def out_dst(row0, nrows):
    return out_hbm.at[pl.ds(row0, nrows), pl.ds(0, d)]

  def drain_scatter(b):
    pltpu.make_async_copy(obuf_v.at[b], out_dst(0, n), ssems[b]).wait()

  for b in range(_NBUF):
    issue_gather(jnp.int32(b), b)

  @pl.loop(0, n_outer)
  def _outer(o):
    for b in range(_NBUF):
      i_local = o * _NBUF + b
      @pl.when(o >= 1)
      def _():
        drain_scatter(b)
      drain_gather(b)

      @plsc.parallel_loop(0, n, unroll=4)
      def _row(j):
        for k in range(d // _LANES):
          sl = pl.ds(k * _LANES, _LANES)
          obuf_v.at[b][j, sl] = rows_v.at[b][j, sl] + pos_v[j, sl]

      @pl.when(o < n_outer - 1)
      def _():
        issue_gather(i_local + _NBUF, b)
      pltpu.async_copy(obuf_v.at[b], out_dst((base_seq + i_local) * n, n),
                       ssems[b])

  for b in range(_NBUF):
    drain_scatter(b)


def kernel(X, token_table, pos_table):
  b, n = X.shape
  v, d = token_table.shape
  assert b % (_NUM_WORKERS * _NBUF) == 0 and d % _LANES == 0
  seq_per_w = b // _NUM_WORKERS
  assert _H0 <= n < 2 * _H0
  n_pad = ((n + _LANES - 1) // _LANES) * _LANES

  xt = X.T.astype(jnp.int32)
  tbl128 = jnp.pad(token_table, ((0, 0), (0, 128 - d)))
  mesh = plsc.VectorSubcoreMesh(core_axis_name="c", subcore_axis_name="s")

  emb = pl.kernel(
      functools.partial(_emb_body, n, d, seq_per_w, n_pad),
      out_type=jax.ShapeDtypeStruct((b * n, 128), jnp.float32),
      mesh=mesh,
      scratch_types=[
          pltpu.VMEM((n_pad, seq_per_w), jnp.int32),
          pltpu.VMEM((_NBUF, n_pad), jnp.int32),
          pltpu.VMEM((_NBUF, n, 128), jnp.float32),
          pltpu.VMEM((_NBUF, n, d), jnp.float32),
          pltpu.VMEM((n, d), jnp.float32),
          pltpu.SemaphoreType.DMA,
          pltpu.SemaphoreType.DMA,
          pltpu.SemaphoreType.DMA,
          pltpu.SemaphoreType.DMA,
      ],
      compiler_params=pltpu.CompilerParams(use_tc_tiling_on_sc=False,
                                           needs_layout_passes=False),
  )
  out = emb(xt, tbl128, pos_table)
  return out[:, :d].reshape(b, n, d)


# speedup vs baseline: 1.4964x; 1.0724x over previous
import functools

import jax
import jax.numpy as jnp
from jax import lax
from jax.experimental import pallas as pl
from jax.experimental.pallas import tpu as pltpu
from jax.experimental.pallas import tpu_sc as plsc

_NUM_CORES = 2
_NUM_SUBCORES = 16
_NUM_WORKERS = _NUM_CORES * _NUM_SUBCORES
_LANES = 16
_NBUF = 2
_H0 = 128


def _emb_body(n, d, seq_per_w, n_pad,
              xt_hbm, tbl_hbm, pos_hbm, out_hbm,
              xbuf_v, gidx_v, rows_v, obuf_v, pos_v,
              gsem0, gsem1, ssem0, ssem1):
  c = lax.axis_index("c")
  s = lax.axis_index("s")
  wid = s * _NUM_CORES + c
  base_seq = wid * seq_per_w
  gsems = (gsem0, gsem1)
  ssems = (ssem0, ssem1)
  n_outer = seq_per_w // _NBUF
  h1 = n - _H0
  nblk = (n + _LANES - 1) // _LANES
  iota = lax.iota(jnp.int32, _LANES)

  pltpu.sync_copy(pos_hbm, pos_v)
  pltpu.sync_copy(xt_hbm.at[:, pl.ds(wid * seq_per_w, seq_per_w)],
                  xbuf_v.at[pl.ds(0, n)])

  def issue_gather(i_local, b):
    col = jnp.full((_LANES,), i_local, jnp.int32)
    for k in range(nblk):
      vals = plsc.load_gather(xbuf_v, [jnp.int32(_LANES * k) + iota, col])
      gidx_v.at[b][pl.ds(_LANES * k, _LANES)] = vals
    rows_b = rows_v.at[b]
    pltpu.async_copy(tbl_hbm.at[gidx_v.at[b, pl.ds(0, _H0)]],
                     rows_b.at[pl.ds(0, _H0)], gsems[b])
    pltpu.async_copy(tbl_hbm.at[gidx_v.at[b, pl.ds(_H0, h1)]],
                     rows_b.at[pl.ds(_H0, h1)], gsems[b])

  def drain_gather(b):
    pltpu.make_async_copy(tbl_hbm.at[pl.ds(0, n)], rows_v.at[b],
                          gsems[b]).wait()

  def out_dst(row0, nrows):
    return out_hbm.at[pl.ds(row0, nrows), pl.ds(0, d)]

  def drain_scatter(b):
    pltpu.make_async_copy(obuf_v.at[b], out_dst(0, n), ssems[b]).wait()

  for b in range(_NBUF):
    issue_gather(jnp.int32(b), b)

  @pl.loop(0, n_outer)
  def _outer(o):
    for b in range(_NBUF):
      i_local = o * _NBUF + b
      @pl.when(o >= 1)
      def _():
        drain_scatter(b)
      drain_gather(b)

      @plsc.parallel_loop(0, n, unroll=4)
      def _row(j):
        for k in range(d // _LANES):
          sl = pl.ds(k * _LANES, _LANES)
          obuf_v.at[b][j, sl] = rows_v.at[b][j, sl] + pos_v[j, sl]

      @pl.when(o < n_outer - 1)
      def _():
        issue_gather(i_local + _NBUF, b)
      pltpu.async_copy(obuf_v.at[b], out_dst((base_seq + i_local) * n, n),
                       ssems[b])

  for b in range(_NBUF):
    drain_scatter(b)


_TCB = 2048


def _tpose_body(d, tin_ref, tout_ref):
  t = jnp.transpose(tin_ref[...], (1, 0))
  tout_ref[...] = jnp.pad(t, ((0, 0), (0, 128 - d)))


def _pad_transpose(tbl_t):
  d, v = tbl_t.shape
  grid = (v + _TCB - 1) // _TCB
  return pl.pallas_call(
      functools.partial(_tpose_body, d),
      grid=(grid,),
      in_specs=[pl.BlockSpec((d, _TCB), lambda i: (0, i))],
      out_specs=pl.BlockSpec((_TCB, 128), lambda i: (i, 0)),
      out_shape=jax.ShapeDtypeStruct((v, 128), jnp.float32),
  )(tbl_t)


def kernel(X, token_table, pos_table):
  b, n = X.shape
  v, d = token_table.shape
  assert b % (_NUM_WORKERS * _NBUF) == 0 and d % _LANES == 0
  seq_per_w = b // _NUM_WORKERS
  assert _H0 <= n < 2 * _H0
  n_pad = ((n + _LANES - 1) // _LANES) * _LANES

  xt = X.T.astype(jnp.int32)
  tbl128 = _pad_transpose(token_table.T)
  mesh = plsc.VectorSubcoreMesh(core_axis_name="c", subcore_axis_name="s")

  emb = pl.kernel(
      functools.partial(_emb_body, n, d, seq_per_w, n_pad),
      out_type=jax.ShapeDtypeStruct((b * n, 128), jnp.float32),
      mesh=mesh,
      scratch_types=[
          pltpu.VMEM((n_pad, seq_per_w), jnp.int32),
          pltpu.VMEM((_NBUF, n_pad), jnp.int32),
          pltpu.VMEM((_NBUF, n, 128), jnp.float32),
          pltpu.VMEM((_NBUF, n, d), jnp.float32),
          pltpu.VMEM((n, d), jnp.float32),
          pltpu.SemaphoreType.DMA,
          pltpu.SemaphoreType.DMA,
          pltpu.SemaphoreType.DMA,
          pltpu.SemaphoreType.DMA,
      ],
      compiler_params=pltpu.CompilerParams(use_tc_tiling_on_sc=False,
                                           needs_layout_passes=False),
  )
  out = emb(xt, tbl128, pos_table)
  return out[:, :d].reshape(b, n, d)


# speedup vs baseline: 1.7499x; 1.1693x over previous
import functools

import jax
import jax.numpy as jnp
from jax import lax
from jax.experimental import pallas as pl
from jax.experimental.pallas import tpu as pltpu
from jax.experimental.pallas import tpu_sc as plsc

_NUM_CORES = 2
_NUM_SUBCORES = 16
_NUM_WORKERS = _NUM_CORES * _NUM_SUBCORES
_LANES = 16
_NBUF = 2
_H0 = 128


def _emb_body(n, d, seq_per_w, n_pad,
              xt_hbm, tbl_hbm, pos_hbm, out_hbm,
              xbuf_v, gidx_v, rows_v, obuf_v, pos_v,
              gsem0, gsem1, ssem0, ssem1):
  c = lax.axis_index("c")
  s = lax.axis_index("s")
  wid = s * _NUM_CORES + c
  base_seq = wid * seq_per_w
  gsems = (gsem0, gsem1)
  ssems = (ssem0, ssem1)
  n_outer = seq_per_w // _NBUF
  h1 = n - _H0
  nblk = (n + _LANES - 1) // _LANES
  iota = lax.iota(jnp.int32, _LANES)

  pltpu.sync_copy(pos_hbm, pos_v)
  pltpu.sync_copy(xt_hbm.at[:, pl.ds(wid * seq_per_w, seq_per_w)],
                  xbuf_v.at[pl.ds(0, n)])

  def issue_gather(i_local, b):
    col = jnp.full((_LANES,), i_local, jnp.int32)
    for k in range(nblk):
      vals = plsc.load_gather(xbuf_v, [jnp.int32(_LANES * k) + iota, col])
      gidx_v.at[b][pl.ds(_LANES * k, _LANES)] = vals
    rows_b = rows_v.at[b]
    pltpu.async_copy(tbl_hbm.at[gidx_v.at[b, pl.ds(0, _H0)]],
                     rows_b.at[pl.ds(0, _H0)], gsems[b])
    pltpu.async_copy(tbl_hbm.at[gidx_v.at[b, pl.ds(_H0, h1)]],
                     rows_b.at[pl.ds(_H0, h1)], gsems[b])

  def drain_gather(b):
    pltpu.make_async_copy(tbl_hbm.at[pl.ds(0, n)], rows_v.at[b],
                          gsems[b]).wait()

  def out_dst(row0, nrows):
    return out_hbm.at[pl.ds(row0, nrows), pl.ds(0, d)]

  def drain_scatter(b):
    pltpu.make_async_copy(obuf_v.at[b], out_dst(0, n), ssems[b]).wait()

  for b in range(_NBUF):
    issue_gather(jnp.int32(b), b)

  @pl.loop(0, n_outer)
  def _outer(o):
    for b in range(_NBUF):
      i_local = o * _NBUF + b
      @pl.when(o >= 1)
      def _():
        drain_scatter(b)
      drain_gather(b)

      @plsc.parallel_loop(0, n, unroll=4)
      def _row(j):
        for k in range(d // _LANES):
          sl = pl.ds(k * _LANES, _LANES)
          obuf_v.at[b][j, sl] = rows_v.at[b][j, sl] + pos_v[j, sl]

      @pl.when(o < n_outer - 1)
      def _():
        issue_gather(i_local + _NBUF, b)
      pltpu.async_copy(obuf_v.at[b], out_dst((base_seq + i_local) * n, n),
                       ssems[b])

  for b in range(_NBUF):
    drain_scatter(b)


_TCB = 4096


def _tpose_body(d, tin_ref, tout_ref):
  tout_ref[:, :d] = jnp.transpose(tin_ref[...], (1, 0))


def _pad_transpose(tbl_t):
  d, v = tbl_t.shape
  grid = (v + _TCB - 1) // _TCB
  return pl.pallas_call(
      functools.partial(_tpose_body, d),
      grid=(grid,),
      in_specs=[pl.BlockSpec((d, _TCB), lambda i: (0, i))],
      out_specs=pl.BlockSpec((_TCB, 128), lambda i: (i, 0)),
      out_shape=jax.ShapeDtypeStruct((v, 128), jnp.float32),
  )(tbl_t)


def kernel(X, token_table, pos_table):
  b, n = X.shape
  v, d = token_table.shape
  assert b % (_NUM_WORKERS * _NBUF) == 0 and d % _LANES == 0
  seq_per_w = b // _NUM_WORKERS
  assert _H0 <= n < 2 * _H0
  n_pad = ((n + _LANES - 1) // _LANES) * _LANES

  xt = X.T.astype(jnp.int32)
  tbl128 = _pad_transpose(token_table.T)
  mesh = plsc.VectorSubcoreMesh(core_axis_name="c", subcore_axis_name="s")

  emb = pl.kernel(
      functools.partial(_emb_body, n, d, seq_per_w, n_pad),
      out_type=jax.ShapeDtypeStruct((b * n, 128), jnp.float32),
      mesh=mesh,
      scratch_types=[
          pltpu.VMEM((n_pad, seq_per_w), jnp.int32),
          pltpu.VMEM((_NBUF, n_pad), jnp.int32),
          pltpu.VMEM((_NBUF, n, 128), jnp.float32),
          pltpu.VMEM((_NBUF, n, d), jnp.float32),
          pltpu.VMEM((n, d), jnp.float32),
          pltpu.SemaphoreType.DMA,
          pltpu.SemaphoreType.DMA,
          pltpu.SemaphoreType.DMA,
          pltpu.SemaphoreType.DMA,
      ],
      compiler_params=pltpu.CompilerParams(use_tc_tiling_on_sc=False,
                                           needs_layout_passes=False),
  )
  out = emb(xt, tbl128, pos_table)
  return out[:, :d].reshape(b, n, d)


# speedup vs baseline: 1.9302x; 1.1031x over previous
import functools

import jax
import jax.numpy as jnp
from jax import lax
from jax.experimental import pallas as pl
from jax.experimental.pallas import tpu as pltpu
from jax.experimental.pallas import tpu_sc as plsc

_NUM_CORES = 2
_NUM_SUBCORES = 16
_NUM_WORKERS = _NUM_CORES * _NUM_SUBCORES
_LANES = 16
_NBUF = 2
_H0 = 128


def _emb_body(n, d, seq_per_w, n_pad,
              xt_hbm, tbl_hbm, pos_hbm, out_hbm,
              xbuf_v, gidx_v, rows_v, obuf_v, pos_v,
              gsem0, gsem1, ssem0, ssem1):
  c = lax.axis_index("c")
  s = lax.axis_index("s")
  wid = s * _NUM_CORES + c
  base_seq = wid * seq_per_w
  gsems = (gsem0, gsem1)
  ssems = (ssem0, ssem1)
  n_outer = seq_per_w // _NBUF
  h1 = n - _H0
  nblk = (n + _LANES - 1) // _LANES
  iota = lax.iota(jnp.int32, _LANES)

  pltpu.sync_copy(pos_hbm, pos_v)
  pltpu.sync_copy(xt_hbm.at[:, pl.ds(wid * seq_per_w, seq_per_w)],
                  xbuf_v.at[pl.ds(0, n)])

  def issue_gather(i_local, b):
    col = jnp.full((_LANES,), i_local, jnp.int32)
    for k in range(nblk):
      vals = plsc.load_gather(xbuf_v, [jnp.int32(_LANES * k) + iota, col])
      gidx_v.at[b][pl.ds(_LANES * k, _LANES)] = vals
    rows_b = rows_v.at[b]
    pltpu.async_copy(tbl_hbm.at[gidx_v.at[b, pl.ds(0, _H0)]],
                     rows_b.at[pl.ds(0, _H0)], gsems[b])
    pltpu.async_copy(tbl_hbm.at[gidx_v.at[b, pl.ds(_H0, h1)]],
                     rows_b.at[pl.ds(_H0, h1)], gsems[b])

  def drain_gather(b):
    pltpu.make_async_copy(tbl_hbm.at[pl.ds(0, n)], rows_v.at[b],
                          gsems[b]).wait()

  def out_dst(row0, nrows):
    return out_hbm.at[pl.ds(row0, nrows), pl.ds(0, d)]

  def drain_scatter(b):
    pltpu.make_async_copy(obuf_v.at[b], out_dst(0, n), ssems[b]).wait()

  for b in range(_NBUF):
    issue_gather(jnp.int32(b), b)

  @pl.loop(0, n_outer)
  def _outer(o):
    for b in range(_NBUF):
      i_local = o * _NBUF + b
      @pl.when(o >= 1)
      def _():
        drain_scatter(b)
      drain_gather(b)

      @plsc.parallel_loop(0, n, unroll=4)
      def _row(j):
        for k in range(d // _LANES):
          sl = pl.ds(k * _LANES, _LANES)
          obuf_v.at[b][j, sl] = rows_v.at[b][j, sl] + pos_v[j, sl]

      @pl.when(o < n_outer - 1)
      def _():
        issue_gather(i_local + _NBUF, b)
      pltpu.async_copy(obuf_v.at[b], out_dst((base_seq + i_local) * n, n),
                       ssems[b])

  for b in range(_NBUF):
    drain_scatter(b)


_TCB = 8192


def _tpose_body(d, tin_ref, tout_ref):
  tout_ref[:, :d] = jnp.transpose(tin_ref[...], (1, 0))


def _pad_transpose(tbl_t):
  d, v = tbl_t.shape
  grid = (v + _TCB - 1) // _TCB
  return pl.pallas_call(
      functools.partial(_tpose_body, d),
      grid=(grid,),
      in_specs=[pl.BlockSpec((d, _TCB), lambda i: (0, i))],
      out_specs=pl.BlockSpec((_TCB, 128), lambda i: (i, 0)),
      out_shape=jax.ShapeDtypeStruct((v, 128), jnp.float32),
  )(tbl_t)


def kernel(X, token_table, pos_table):
  b, n = X.shape
  v, d = token_table.shape
  assert b % (_NUM_WORKERS * _NBUF) == 0 and d % _LANES == 0
  seq_per_w = b // _NUM_WORKERS
  assert _H0 <= n < 2 * _H0
  n_pad = ((n + _LANES - 1) // _LANES) * _LANES

  xt = X.T.astype(jnp.int32)
  tbl128 = _pad_transpose(token_table.T)
  mesh = plsc.VectorSubcoreMesh(core_axis_name="c", subcore_axis_name="s")

  emb = pl.kernel(
      functools.partial(_emb_body, n, d, seq_per_w, n_pad),
      out_type=jax.ShapeDtypeStruct((b * n, 128), jnp.float32),
      mesh=mesh,
      scratch_types=[
          pltpu.VMEM((n_pad, seq_per_w), jnp.int32),
          pltpu.VMEM((_NBUF, n_pad), jnp.int32),
          pltpu.VMEM((_NBUF, n, 128), jnp.float32),
          pltpu.VMEM((_NBUF, n, d), jnp.float32),
          pltpu.VMEM((n, d), jnp.float32),
          pltpu.SemaphoreType.DMA,
          pltpu.SemaphoreType.DMA,
          pltpu.SemaphoreType.DMA,
          pltpu.SemaphoreType.DMA,
      ],
      compiler_params=pltpu.CompilerParams(use_tc_tiling_on_sc=False,
                                           needs_layout_passes=False),
  )
  out = emb(xt, tbl128, pos_table)
  return out[:, :d].reshape(b, n, d)


# speedup vs baseline: 1.9848x; 1.0283x over previous
import functools

import jax
import jax.numpy as jnp
from jax import lax
from jax.experimental import pallas as pl
from jax.experimental.pallas import tpu as pltpu
from jax.experimental.pallas import tpu_sc as plsc

_NUM_CORES = 2
_NUM_SUBCORES = 16
_NUM_WORKERS = _NUM_CORES * _NUM_SUBCORES
_LANES = 16
_NBUF = 2
_H0 = 128


def _emb_body(n, d, seq_per_w, n_pad,
              xt_hbm, tbl_hbm, pos_hbm, out_hbm,
              xbuf_v, gidx_v, rows_v, obuf_v, pos_v,
              gsem0, gsem1, ssem0, ssem1):
  c = lax.axis_index("c")
  s = lax.axis_index("s")
  wid = s * _NUM_CORES + c
  base_seq = wid * seq_per_w
  gsems = (gsem0, gsem1)
  ssems = (ssem0, ssem1)
  n_outer = seq_per_w // _NBUF
  h1 = n - _H0
  nblk = (n + _LANES - 1) // _LANES
  iota = lax.iota(jnp.int32, _LANES)

  pltpu.sync_copy(pos_hbm, pos_v)
  pltpu.sync_copy(xt_hbm.at[:, pl.ds(wid * seq_per_w, seq_per_w)],
                  xbuf_v.at[pl.ds(0, n)])

  def issue_gather(i_local, b):
    col = jnp.full((_LANES,), i_local, jnp.int32)
    for k in range(nblk):
      vals = plsc.load_gather(xbuf_v, [jnp.int32(_LANES * k) + iota, col])
      gidx_v.at[b][pl.ds(_LANES * k, _LANES)] = vals
    rows_b = rows_v.at[b]
    pltpu.async_copy(tbl_hbm.at[gidx_v.at[b, pl.ds(0, _H0)]],
                     rows_b.at[pl.ds(0, _H0)], gsems[b])
    pltpu.async_copy(tbl_hbm.at[gidx_v.at[b, pl.ds(_H0, h1)]],
                     rows_b.at[pl.ds(_H0, h1)], gsems[b])

  def drain_gather(b):
    pltpu.make_async_copy(tbl_hbm.at[pl.ds(0, n)], rows_v.at[b],
                          gsems[b]).wait()

  def out_dst(row0, nrows):
    return out_hbm.at[pl.ds(row0, nrows), pl.ds(0, d)]

  def drain_scatter(b):
    pltpu.make_async_copy(obuf_v.at[b], out_dst(0, n), ssems[b]).wait()

  for b in range(_NBUF):
    issue_gather(jnp.int32(b), b)

  @pl.loop(0, n_outer)
  def _outer(o):
    for b in range(_NBUF):
      i_local = o * _NBUF + b
      @pl.when(o >= 1)
      def _():
        drain_scatter(b)
      drain_gather(b)

      @plsc.parallel_loop(0, n, unroll=4)
      def _row(j):
        for k in range(d // _LANES):
          sl = pl.ds(k * _LANES, _LANES)
          obuf_v.at[b][j, sl] = rows_v.at[b][j, sl] + pos_v[j, sl]

      @pl.when(o < n_outer - 1)
      def _():
        issue_gather(i_local + _NBUF, b)
      pltpu.async_copy(obuf_v.at[b], out_dst((base_seq + i_local) * n, n),
                       ssems[b])

  for b in range(_NBUF):
    drain_scatter(b)


_TCB = 16384


def _tpose_body(d, tin_ref, tout_ref):
  tout_ref[:, :d] = jnp.transpose(tin_ref[...], (1, 0))


def _pad_transpose(tbl_t):
  d, v = tbl_t.shape
  grid = (v + _TCB - 1) // _TCB
  return pl.pallas_call(
      functools.partial(_tpose_body, d),
      grid=(grid,),
      in_specs=[pl.BlockSpec((d, _TCB), lambda i: (0, i))],
      out_specs=pl.BlockSpec((_TCB, 128), lambda i: (i, 0)),
      out_shape=jax.ShapeDtypeStruct((v, 128), jnp.float32),
  )(tbl_t)


def kernel(X, token_table, pos_table):
  b, n = X.shape
  v, d = token_table.shape
  assert b % (_NUM_WORKERS * _NBUF) == 0 and d % _LANES == 0
  seq_per_w = b // _NUM_WORKERS
  assert _H0 <= n < 2 * _H0
  n_pad = ((n + _LANES - 1) // _LANES) * _LANES

  xt = X.T.astype(jnp.int32)
  tbl128 = _pad_transpose(token_table.T)
  mesh = plsc.VectorSubcoreMesh(core_axis_name="c", subcore_axis_name="s")

  emb = pl.kernel(
      functools.partial(_emb_body, n, d, seq_per_w, n_pad),
      out_type=jax.ShapeDtypeStruct((b * n, 128), jnp.float32),
      mesh=mesh,
      scratch_types=[
          pltpu.VMEM((n_pad, seq_per_w), jnp.int32),
          pltpu.VMEM((_NBUF, n_pad), jnp.int32),
          pltpu.VMEM((_NBUF, n, 128), jnp.float32),
          pltpu.VMEM((_NBUF, n, d), jnp.float32),
          pltpu.VMEM((n, d), jnp.float32),
          pltpu.SemaphoreType.DMA,
          pltpu.SemaphoreType.DMA,
          pltpu.SemaphoreType.DMA,
          pltpu.SemaphoreType.DMA,
      ],
      compiler_params=pltpu.CompilerParams(use_tc_tiling_on_sc=False,
                                           needs_layout_passes=False),
  )
  out = emb(xt, tbl128, pos_table)
  return out[:, :d].reshape(b, n, d)


# speedup vs baseline: 2.0096x; 1.0125x over previous
import functools

import jax
import jax.numpy as jnp
from jax import lax
from jax.experimental import pallas as pl
from jax.experimental.pallas import tpu as pltpu
from jax.experimental.pallas import tpu_sc as plsc

_NUM_CORES = 2
_NUM_SUBCORES = 16
_NUM_WORKERS = _NUM_CORES * _NUM_SUBCORES
_LANES = 16
_NBUF = 2
_H0 = 128


def _emb_body(n, d, seq_per_w, n_pad,
              xt_hbm, tbl_hbm, pos_hbm, out_hbm,
              xbuf_v, gidx_v, rows_v, obuf_v, pos_v,
              gsem0, gsem1, ssem0, ssem1):
  c = lax.axis_index("c")
  s = lax.axis_index("s")
  wid = s * _NUM_CORES + c
  base_seq = wid * seq_per_w
  gsems = (gsem0, gsem1)
  ssems = (ssem0, ssem1)
  n_outer = seq_per_w // _NBUF
  h1 = n - _H0
  nblk = (n + _LANES - 1) // _LANES
  iota = lax.iota(jnp.int32, _LANES)

  pltpu.sync_copy(pos_hbm, pos_v)
  pltpu.sync_copy(xt_hbm.at[:, pl.ds(wid * seq_per_w, seq_per_w)],
                  xbuf_v.at[pl.ds(0, n)])

  def issue_gather(i_local, b):
    col = jnp.full((_LANES,), i_local, jnp.int32)
    for k in range(nblk):
      vals = plsc.load_gather(xbuf_v, [jnp.int32(_LANES * k) + iota, col])
      gidx_v.at[b][pl.ds(_LANES * k, _LANES)] = vals
    rows_b = rows_v.at[b]
    pltpu.async_copy(tbl_hbm.at[gidx_v.at[b, pl.ds(0, _H0)]],
                     rows_b.at[pl.ds(0, _H0)], gsems[b])
    pltpu.async_copy(tbl_hbm.at[gidx_v.at[b, pl.ds(_H0, h1)]],
                     rows_b.at[pl.ds(_H0, h1)], gsems[b])

  def drain_gather(b):
    pltpu.make_async_copy(tbl_hbm.at[pl.ds(0, n)], rows_v.at[b],
                          gsems[b]).wait()

  def out_dst(row0, nrows):
    return out_hbm.at[pl.ds(row0, nrows), pl.ds(0, d)]

  def drain_scatter(b):
    pltpu.make_async_copy(obuf_v.at[b], out_dst(0, n), ssems[b]).wait()

  for b in range(_NBUF):
    issue_gather(jnp.int32(b), b)

  @pl.loop(0, n_outer)
  def _outer(o):
    for b in range(_NBUF):
      i_local = o * _NBUF + b
      @pl.when(o >= 1)
      def _():
        drain_scatter(b)
      drain_gather(b)

      @plsc.parallel_loop(0, n, unroll=4)
      def _row(j):
        for k in range(d // _LANES):
          sl = pl.ds(k * _LANES, _LANES)
          obuf_v.at[b][j, sl] = rows_v.at[b][j, sl] + pos_v[j, sl]

      @pl.when(o < n_outer - 1)
      def _():
        issue_gather(i_local + _NBUF, b)
      pltpu.async_copy(obuf_v.at[b], out_dst((base_seq + i_local) * n, n),
                       ssems[b])

  for b in range(_NBUF):
    drain_scatter(b)


_TCB = 32768


def _tpose_body(d, tin_ref, tout_ref):
  tout_ref[:, :d] = jnp.transpose(tin_ref[...], (1, 0))


def _pad_transpose(tbl_t):
  d, v = tbl_t.shape
  grid = (v + _TCB - 1) // _TCB
  return pl.pallas_call(
      functools.partial(_tpose_body, d),
      grid=(grid,),
      in_specs=[pl.BlockSpec((d, _TCB), lambda i: (0, i))],
      out_specs=pl.BlockSpec((_TCB, 128), lambda i: (i, 0)),
      out_shape=jax.ShapeDtypeStruct((v, 128), jnp.float32),
  )(tbl_t)


def kernel(X, token_table, pos_table):
  b, n = X.shape
  v, d = token_table.shape
  assert b % (_NUM_WORKERS * _NBUF) == 0 and d % _LANES == 0
  seq_per_w = b // _NUM_WORKERS
  assert _H0 <= n < 2 * _H0
  n_pad = ((n + _LANES - 1) // _LANES) * _LANES

  xt = X.T.astype(jnp.int32)
  tbl128 = _pad_transpose(token_table.T)
  mesh = plsc.VectorSubcoreMesh(core_axis_name="c", subcore_axis_name="s")

  emb = pl.kernel(
      functools.partial(_emb_body, n, d, seq_per_w, n_pad),
      out_type=jax.ShapeDtypeStruct((b * n, 128), jnp.float32),
      mesh=mesh,
      scratch_types=[
          pltpu.VMEM((n_pad, seq_per_w), jnp.int32),
          pltpu.VMEM((_NBUF, n_pad), jnp.int32),
          pltpu.VMEM((_NBUF, n, 128), jnp.float32),
          pltpu.VMEM((_NBUF, n, d), jnp.float32),
          pltpu.VMEM((n, d), jnp.float32),
          pltpu.SemaphoreType.DMA,
          pltpu.SemaphoreType.DMA,
          pltpu.SemaphoreType.DMA,
          pltpu.SemaphoreType.DMA,
      ],
      compiler_params=pltpu.CompilerParams(use_tc_tiling_on_sc=False,
                                           needs_layout_passes=False),
  )
  out = emb(xt, tbl128, pos_table)
  return out[:, :d].reshape(b, n, d)
